# baseline (speedup 1.0000x reference)
import jax
import jax.numpy as jnp
from jax import lax
from jax.experimental import pallas as pl
from jax.experimental.pallas import tpu as pltpu
from jax.experimental.pallas import tpu_sc as plsc

N = 163842
E = 983040
H2 = 82048
NP = 2 * H2
RPT = H2 // 16
NF = NP // 8
ET = E // 16
C = 2048
NCHUNK = ET // C
G = 128
EB = E // 512


def _prep_body(evx_ref, evy_ref, mu_ref, csig_ref,
               a0_ref, a1_ref, a2_ref, b0_ref, b1_ref, b2_ref):
    ex = evx_ref[...]
    ey = evy_ref[...]
    outs = ((a0_ref, a1_ref, a2_ref), (b0_ref, b1_ref, b2_ref))
    for li in range(2):
        for k in range(3):
            dx = ex - mu_ref[li, k, 0]
            dy = ey - mu_ref[li, k, 1]
            outs[li][k][...] = jnp.exp(dx * dx * csig_ref[li, k, 0]
                                       + dy * dy * csig_ref[li, k, 1])


def _prep(evx, evy, mu_all, csig_all):
    rb = 8
    sds = jax.ShapeDtypeStruct((EB, 512), jnp.float32)
    bs = pl.BlockSpec((rb, 512), lambda i: (i, 0))
    return pl.pallas_call(
        _prep_body,
        grid=(EB // rb,),
        in_specs=[
            bs, bs,
            pl.BlockSpec(memory_space=pltpu.SMEM),
            pl.BlockSpec(memory_space=pltpu.SMEM),
        ],
        out_specs=[bs] * 6,
        out_shape=[sds] * 6,
    )(evx, evy, mu_all, csig_all)


def _dense_body(x_ref, g_ref, root_ref, y_ref, r_ref):
    x = x_ref[...]
    y_ref[...] = jnp.dot(x, g_ref[...], preferred_element_type=jnp.float32)
    r_ref[...] = jnp.dot(x, root_ref[...], preferred_element_type=jnp.float32)


def _dense(x_fold, g_big, root_big):
    blk = 2048
    return pl.pallas_call(
        _dense_body,
        grid=(pl.cdiv(NF, blk),),
        in_specs=[
            pl.BlockSpec((blk, 128), lambda i: (i, 0)),
            pl.BlockSpec((128, 384), lambda i: (0, 0)),
            pl.BlockSpec((128, 128), lambda i: (0, 0)),
        ],
        out_specs=[
            pl.BlockSpec((blk, 384), lambda i: (i, 0)),
            pl.BlockSpec((blk, 128), lambda i: (i, 0)),
        ],
        out_shape=[
            jax.ShapeDtypeStruct((NF, 384), jnp.float32),
            jax.ShapeDtypeStruct((NF, 128), jnp.float32),
        ],
    )(x_fold, g_big, root_big)


def _mid_body(aggm_ref, r_ref, bias_ref, g_ref, root_ref, y_ref, r2_ref):
    x = jnp.maximum(aggm_ref[...] + r_ref[...] + bias_ref[...], 0.0)
    y_ref[...] = jnp.dot(x, g_ref[...], preferred_element_type=jnp.float32)
    r2_ref[...] = jnp.dot(x, root_ref[...], preferred_element_type=jnp.float32)


def _mid(aggm_fold, r_fold, bias_big, g_big, root_big):
    blk = 2048
    return pl.pallas_call(
        _mid_body,
        grid=(pl.cdiv(NF, blk),),
        in_specs=[
            pl.BlockSpec((blk, 128), lambda i: (i, 0)),
            pl.BlockSpec((blk, 128), lambda i: (i, 0)),
            pl.BlockSpec((128,), lambda i: (0,)),
            pl.BlockSpec((128, 384), lambda i: (0, 0)),
            pl.BlockSpec((128, 128), lambda i: (0, 0)),
        ],
        out_specs=[
            pl.BlockSpec((blk, 384), lambda i: (i, 0)),
            pl.BlockSpec((blk, 128), lambda i: (i, 0)),
        ],
        out_shape=[
            jax.ShapeDtypeStruct((NF, 384), jnp.float32),
            jax.ShapeDtypeStruct((NF, 128), jnp.float32),
        ],
    )(aggm_fold, r_fold, bias_big, g_big, root_big)


def _final_body(aggm_ref, r_ref, bias_ref, w_ref, b_ref, ls_ref, lg_ref):
    x = jnp.maximum(aggm_ref[...] + r_ref[...] + bias_ref[...], 0.0)
    ll = jnp.dot(x, w_ref[...], preferred_element_type=jnp.float32) + b_ref[...]
    lo = ll[:, 0:16]
    lsw = ll[:, 16:32]
    mx = jnp.maximum(lo, lsw)
    lse = mx + jnp.log(jnp.exp(lo - mx) + jnp.exp(lsw - mx))
    ls_ref[...] = lo - lse
    lg_ref[...] = lo


def _final(aggm_fold, r_fold, bias_big, w_big2, b_big2):
    blk = 2048
    return pl.pallas_call(
        _final_body,
        grid=(pl.cdiv(NF, blk),),
        in_specs=[
            pl.BlockSpec((blk, 128), lambda i: (i, 0)),
            pl.BlockSpec((blk, 128), lambda i: (i, 0)),
            pl.BlockSpec((128,), lambda i: (0,)),
            pl.BlockSpec((128, 32), lambda i: (0, 0)),
            pl.BlockSpec((32,), lambda i: (0,)),
        ],
        out_specs=[
            pl.BlockSpec((blk, 16), lambda i: (i, 0)),
            pl.BlockSpec((blk, 16), lambda i: (i, 0)),
        ],
        out_shape=[
            jax.ShapeDtypeStruct((NF, 16), jnp.float32),
            jax.ShapeDtypeStruct((NF, 16), jnp.float32),
        ],
    )(aggm_fold, r_fold, bias_big, w_big2, b_big2)


def _sc_body(y_hbm, src_hbm, dst_hbm, gw0_hbm, gw1_hbm, gw2_hbm, aggm_out,
             stage_dst, stage_src, stage_gw0, stage_gw1, stage_gw2,
             sidx0, sidx1, mfb0, mfb1, rows0, rows1, msg0, msg1,
             wbuf, cbuf,
             acc, acc_cnt, gs0, gs1, ss0, ss1, cs0, cs1):
    c = lax.axis_index("c")
    s = lax.axis_index("s")
    base = c * H2

    zf = jnp.zeros((16,), jnp.float32)

    def fill16(i, _):
        wbuf[i, :] = zf
        return 0
    lax.fori_loop(0, 512, fill16, 0)

    def fill1(i, _):
        cbuf[pl.ds(i * 16, 16)] = zf
        return 0
    lax.fori_loop(0, 32, fill1, 0)

    def zacc(j, _):
        pltpu.sync_copy(wbuf, acc.at[pl.ds(s * RPT + j * 512, 512)])
        pltpu.sync_copy(cbuf, acc_cnt.at[pl.ds(s * RPT + j * 512, 512)])
        return 0
    lax.fori_loop(0, 10, zacc, 0)
    pltpu.sync_copy(wbuf.at[pl.ds(0, 8)], acc.at[pl.ds(s * RPT + 5120, 8)])
    pltpu.sync_copy(cbuf.at[pl.ds(0, 8)], acc_cnt.at[pl.ds(s * RPT + 5120, 8)])

    plsc.subcore_barrier()

    def _compute_block(boff, rows, msg, sidx, mfb):
        def ed16(q, _):
            off = boff + q * 16
            d = stage_dst[pl.ds(off, 16)]
            dl = d - base
            inr = (dl >= 0) & (dl < H2)
            mf = jnp.where(inr, 1.0, 0.0).astype(jnp.float32)
            sidx[pl.ds(q * 16, 16)] = jnp.where(inr, dl, 0)
            mfb[pl.ds(q * 16, 16)] = mf
            r = off >> 9
            cc = off & 511
            w0v = stage_gw0[r, pl.ds(cc, 16)] * mf
            w1v = stage_gw1[r, pl.ds(cc, 16)] * mf
            w2v = stage_gw2[r, pl.ds(cc, 16)] * mf
            for j in range(16):
                i = q * 16 + j
                mv = (rows[i, pl.ds(0, 16)] * w0v[j]
                      + rows[i, pl.ds(16, 16)] * w1v[j]
                      + rows[i, pl.ds(32, 16)] * w2v[j])
                msg[i, :] = mv
            return 0
        lax.fori_loop(0, G // 16, ed16, 0)

    def _drain_scatters():
        pltpu.make_async_copy(msg0, acc.at[sidx0], ss0).wait()
        pltpu.make_async_copy(mfb0, acc_cnt.at[sidx0], cs0).wait()
        pltpu.make_async_copy(msg1, acc.at[sidx1], ss1).wait()
        pltpu.make_async_copy(mfb1, acc_cnt.at[sidx1], cs1).wait()

    NB = C // G

    def chunk_body(ch, _):
        e0 = s * ET + ch * C
        r0w = e0 >> 9
        pltpu.sync_copy(dst_hbm.at[pl.ds(e0, C)], stage_dst)
        pltpu.sync_copy(src_hbm.at[pl.ds(e0, C)], stage_src)
        pltpu.sync_copy(gw0_hbm.at[pl.ds(r0w, C // 512)], stage_gw0)
        pltpu.sync_copy(gw1_hbm.at[pl.ds(r0w, C // 512)], stage_gw1)
        pltpu.sync_copy(gw2_hbm.at[pl.ds(r0w, C // 512)], stage_gw2)

        pltpu.async_copy(y_hbm.at[stage_src.at[pl.ds(0, G)]], rows0, gs0)

        def pair(p, _):
            b0 = 2 * p * G
            b1 = b0 + G
            pltpu.async_copy(y_hbm.at[stage_src.at[pl.ds(b1, G)]], rows1, gs1)
            pltpu.make_async_copy(y_hbm.at[stage_src.at[pl.ds(0, G)]], rows0, gs0).wait()

            @pl.when(p > 0)
            def _():
                pltpu.make_async_copy(msg0, acc.at[sidx0], ss0).wait()
                pltpu.make_async_copy(mfb0, acc_cnt.at[sidx0], cs0).wait()

            _compute_block(b0, rows0, msg0, sidx0, mfb0)
            pltpu.async_copy(msg0, acc.at[sidx0], ss0, add=True)
            pltpu.async_copy(mfb0, acc_cnt.at[sidx0], cs0, add=True)

            @pl.when(p < NB // 2 - 1)
            def _():
                pltpu.async_copy(y_hbm.at[stage_src.at[pl.ds(b1 + G, G)]], rows0, gs0)

            pltpu.make_async_copy(y_hbm.at[stage_src.at[pl.ds(0, G)]], rows1, gs1).wait()

            @pl.when(p > 0)
            def _():
                pltpu.make_async_copy(msg1, acc.at[sidx1], ss1).wait()
                pltpu.make_async_copy(mfb1, acc_cnt.at[sidx1], cs1).wait()

            _compute_block(b1, rows1, msg1, sidx1, mfb1)
            pltpu.async_copy(msg1, acc.at[sidx1], ss1, add=True)
            pltpu.async_copy(mfb1, acc_cnt.at[sidx1], cs1, add=True)
            return 0

        lax.fori_loop(0, NB // 2, pair, 0)
        _drain_scatters()
        return 0

    lax.fori_loop(0, NCHUNK, chunk_body, 0)

    plsc.subcore_barrier()

    def wblock(jb, _):
        r0 = s * RPT + jb * 512
        pltpu.sync_copy(acc.at[pl.ds(r0, 512)], wbuf)
        pltpu.sync_copy(acc_cnt.at[pl.ds(r0, 512)], cbuf)

        def wg(q, _):
            cv = jnp.maximum(cbuf[pl.ds(q * 16, 16)], 1.0)
            for j in range(16):
                i = q * 16 + j
                wbuf[i, :] = wbuf[i, :] / cv[j]
            return 0
        lax.fori_loop(0, 32, wg, 0)
        pltpu.sync_copy(wbuf, aggm_out.at[c, pl.ds(r0, 512)])
        return 0
    lax.fori_loop(0, 10, wblock, 0)

    r0 = s * RPT + 5120
    pltpu.sync_copy(acc.at[pl.ds(r0, 8)], wbuf.at[pl.ds(0, 8)])
    pltpu.sync_copy(acc_cnt.at[pl.ds(r0, 8)], cbuf.at[pl.ds(0, 8)])
    cv = jnp.maximum(cbuf[pl.ds(0, 16)], 1.0)
    for j in range(8):
        wbuf[j, :] = wbuf[j, :] / cv[j]
    pltpu.sync_copy(wbuf.at[pl.ds(0, 8)], aggm_out.at[c, pl.ds(r0, 8)])


_sc_layer = pl.kernel(
    _sc_body,
    out_type=jax.ShapeDtypeStruct((2, H2, 16), jnp.float32),
    mesh=plsc.VectorSubcoreMesh(core_axis_name="c", subcore_axis_name="s"),
    compiler_params=pltpu.CompilerParams(use_tc_tiling_on_sc=False),
    scratch_types=[
        pltpu.VMEM((C,), jnp.int32),
        pltpu.VMEM((C,), jnp.int32),
        pltpu.VMEM((C // 512, 512), jnp.float32),
        pltpu.VMEM((C // 512, 512), jnp.float32),
        pltpu.VMEM((C // 512, 512), jnp.float32),
        pltpu.VMEM((G,), jnp.int32),
        pltpu.VMEM((G,), jnp.int32),
        pltpu.VMEM((G,), jnp.float32),
        pltpu.VMEM((G,), jnp.float32),
        pltpu.VMEM((G, 48), jnp.float32),
        pltpu.VMEM((G, 48), jnp.float32),
        pltpu.VMEM((G, 16), jnp.float32),
        pltpu.VMEM((G, 16), jnp.float32),
        pltpu.VMEM((512, 16), jnp.float32),
        pltpu.VMEM((512,), jnp.float32),
        pltpu.VMEM_SHARED((H2, 16), jnp.float32),
        pltpu.VMEM_SHARED((H2,), jnp.float32),
        pltpu.SemaphoreType.DMA,
        pltpu.SemaphoreType.DMA,
        pltpu.SemaphoreType.DMA,
        pltpu.SemaphoreType.DMA,
        pltpu.SemaphoreType.DMA,
        pltpu.SemaphoreType.DMA,
    ],
)


def kernel(data, edges, edge_vectors, g0, mu0, sigma0, root0, bias0,
           g1, mu1, sigma1, root1, bias1, fc_w, fc_b):
    f32 = jnp.float32
    src = edges[0]
    dst = edges[1]
    evx = edge_vectors[:, 0].reshape(EB, 512)
    evy = edge_vectors[:, 1].reshape(EB, 512)
    data_p = jnp.pad(data, ((0, NP - N), (0, 0)))
    x_fold = data_p.reshape(NF, 128)

    eye8 = jnp.eye(8, dtype=f32)
    gb0 = jnp.kron(eye8, g0)
    rb0 = jnp.kron(eye8, root0)
    gb1 = jnp.kron(eye8, g1)
    rb1 = jnp.kron(eye8, root1)
    bb0 = jnp.tile(bias0, 8)
    bb1 = jnp.tile(bias1, 8)
    wb2 = jnp.concatenate([jnp.kron(eye8, fc_w), jnp.kron(eye8, fc_w[:, ::-1])], axis=1)
    fb2 = jnp.concatenate([jnp.tile(fc_b, 8), jnp.tile(fc_b[::-1], 8)])

    mu_all = jnp.stack([mu0, mu1]).astype(f32)
    csig_all = -0.5 / (1e-15 + jnp.stack([sigma0, sigma1]).astype(f32) ** 2)

    ga0, ga1, ga2, gc0, gc1, gc2 = _prep(evx, evy, mu_all, csig_all)

    y1, r1 = _dense(x_fold, gb0, rb0)
    aggm1 = _sc_layer(y1.reshape(NP, 48), src, dst, ga0, ga1, ga2)
    aggm1_fold = aggm1.reshape(NF, 128)

    y2, r2 = _mid(aggm1_fold, r1, bb0, gb1, rb1)
    aggm2 = _sc_layer(y2.reshape(NP, 48), src, dst, gc0, gc1, gc2)
    aggm2_fold = aggm2.reshape(NF, 128)

    ls, lg = _final(aggm2_fold, r2, bb1, wb2, fb2)
    log_sm = ls.reshape(NP, 2)[:N]
    nll = lg.reshape(NP, 2)[:N, 0:1]
    return (log_sm, nll)


# speedup vs baseline: 1.2252x; 1.2252x over previous
import jax
import jax.numpy as jnp
from jax import lax
from jax.experimental import pallas as pl
from jax.experimental.pallas import tpu as pltpu
from jax.experimental.pallas import tpu_sc as plsc

N = 163842
E = 983040
H2 = 82048
NP = 2 * H2
RPT = H2 // 16
NF = NP // 8
ET = E // 16
C = 2048
NCHUNK = ET // C
NCHG = E // C
G = 128


def _dense_body(x_ref, g_ref, root_ref, y_ref, r_ref):
    x = x_ref[...]
    y_ref[...] = jnp.dot(x, g_ref[...], preferred_element_type=jnp.float32)
    r_ref[...] = jnp.dot(x, root_ref[...], preferred_element_type=jnp.float32)


def _dense(x_fold, g_big, root_big):
    blk = 2048
    return pl.pallas_call(
        _dense_body,
        grid=(pl.cdiv(NF, blk),),
        in_specs=[
            pl.BlockSpec((blk, 128), lambda i: (i, 0)),
            pl.BlockSpec((128, 384), lambda i: (0, 0)),
            pl.BlockSpec((128, 128), lambda i: (0, 0)),
        ],
        out_specs=[
            pl.BlockSpec((blk, 384), lambda i: (i, 0)),
            pl.BlockSpec((blk, 128), lambda i: (i, 0)),
        ],
        out_shape=[
            jax.ShapeDtypeStruct((NF, 384), jnp.float32),
            jax.ShapeDtypeStruct((NF, 128), jnp.float32),
        ],
    )(x_fold, g_big, root_big)


def _mid_body(aggm_ref, r_ref, bias_ref, g_ref, root_ref, y_ref, r2_ref):
    x = jnp.maximum(aggm_ref[...] + r_ref[...] + bias_ref[...], 0.0)
    y_ref[...] = jnp.dot(x, g_ref[...], preferred_element_type=jnp.float32)
    r2_ref[...] = jnp.dot(x, root_ref[...], preferred_element_type=jnp.float32)


def _mid(aggm_fold, r_fold, bias_big, g_big, root_big):
    blk = 2048
    return pl.pallas_call(
        _mid_body,
        grid=(pl.cdiv(NF, blk),),
        in_specs=[
            pl.BlockSpec((blk, 128), lambda i: (i, 0)),
            pl.BlockSpec((blk, 128), lambda i: (i, 0)),
            pl.BlockSpec((128,), lambda i: (0,)),
            pl.BlockSpec((128, 384), lambda i: (0, 0)),
            pl.BlockSpec((128, 128), lambda i: (0, 0)),
        ],
        out_specs=[
            pl.BlockSpec((blk, 384), lambda i: (i, 0)),
            pl.BlockSpec((blk, 128), lambda i: (i, 0)),
        ],
        out_shape=[
            jax.ShapeDtypeStruct((NF, 384), jnp.float32),
            jax.ShapeDtypeStruct((NF, 128), jnp.float32),
        ],
    )(aggm_fold, r_fold, bias_big, g_big, root_big)


def _final_body(aggm_ref, r_ref, bias_ref, w_ref, b_ref, ls_ref, lg_ref):
    x = jnp.maximum(aggm_ref[...] + r_ref[...] + bias_ref[...], 0.0)
    ll = jnp.dot(x, w_ref[...], preferred_element_type=jnp.float32) + b_ref[...]
    lo = ll[:, 0:16]
    lsw = ll[:, 16:32]
    mx = jnp.maximum(lo, lsw)
    lse = mx + jnp.log(jnp.exp(lo - mx) + jnp.exp(lsw - mx))
    ls_ref[...] = lo - lse
    lg_ref[...] = lo


def _final(aggm_fold, r_fold, bias_big, w_big2, b_big2):
    blk = 2048
    return pl.pallas_call(
        _final_body,
        grid=(pl.cdiv(NF, blk),),
        in_specs=[
            pl.BlockSpec((blk, 128), lambda i: (i, 0)),
            pl.BlockSpec((blk, 128), lambda i: (i, 0)),
            pl.BlockSpec((128,), lambda i: (0,)),
            pl.BlockSpec((128, 32), lambda i: (0, 0)),
            pl.BlockSpec((32,), lambda i: (0,)),
        ],
        out_specs=[
            pl.BlockSpec((blk, 16), lambda i: (i, 0)),
            pl.BlockSpec((blk, 16), lambda i: (i, 0)),
        ],
        out_shape=[
            jax.ShapeDtypeStruct((NF, 16), jnp.float32),
            jax.ShapeDtypeStruct((NF, 16), jnp.float32),
        ],
    )(aggm_fold, r_fold, bias_big, w_big2, b_big2)


def _make_sc_body(with_cnt):
    def body(*args):
        if with_cnt:
            (y_hbm, edata_hbm, prm_hbm, aggm_out, cnt_out,
             stage_all, sidx0, sidx1, mfb0, mfb1, rows0, rows1, msg0, msg1,
             pbuf, wbuf, cbuf, acc, acc_cnt,
             gs0, gs1, ss0, ss1, cs0, cs1) = args
        else:
            (y_hbm, edata_hbm, prm_hbm, cnt_hbm, aggm_out,
             stage_all, sidx0, sidx1, rows0, rows1, msg0, msg1,
             pbuf, wbuf, cbuf, acc,
             gs0, gs1, ss0, ss1) = args
        c = lax.axis_index("c")
        s = lax.axis_index("s")
        base = c * H2

        pltpu.sync_copy(prm_hbm, pbuf)
        pv = pbuf[pl.ds(0, 16)]
        mx0, my0, mx1, my1, mx2, my2 = pv[0], pv[1], pv[2], pv[3], pv[4], pv[5]
        cx0, cy0, cx1, cy1, cx2, cy2 = pv[6], pv[7], pv[8], pv[9], pv[10], pv[11]

        zf = jnp.zeros((16,), jnp.float32)

        def fill16(i, _):
            wbuf[i, :] = zf
            return 0
        lax.fori_loop(0, 512, fill16, 0)

        def fill1(i, _):
            cbuf[pl.ds(i * 16, 16)] = zf
            return 0
        lax.fori_loop(0, 32, fill1, 0)

        def zacc(j, _):
            pltpu.sync_copy(wbuf, acc.at[pl.ds(s * RPT + j * 512, 512)])
            if with_cnt:
                pltpu.sync_copy(cbuf, acc_cnt.at[pl.ds(s * RPT + j * 512, 512)])
            return 0
        lax.fori_loop(0, 10, zacc, 0)
        pltpu.sync_copy(wbuf.at[pl.ds(0, 8)], acc.at[pl.ds(s * RPT + 5120, 8)])
        if with_cnt:
            pltpu.sync_copy(cbuf.at[pl.ds(0, 8)],
                            acc_cnt.at[pl.ds(s * RPT + 5120, 8)])

        plsc.subcore_barrier()

        def _compute_block(boff, rows, msg, sidx, mfb):
            def ed16(q, _):
                off = boff + q * 16
                d = stage_all[0, pl.ds(off, 16)]
                dl = d - base
                inr = (dl >= 0) & (dl < H2)
                mf = jnp.where(inr, 1.0, 0.0).astype(jnp.float32)
                sidx[pl.ds(q * 16, 16)] = jnp.where(inr, dl, 0)
                if with_cnt:
                    mfb[pl.ds(q * 16, 16)] = mf
                ex = plsc.bitcast(stage_all[2, pl.ds(off, 16)], jnp.float32)
                ey = plsc.bitcast(stage_all[3, pl.ds(off, 16)], jnp.float32)
                dx = ex - mx0
                dy = ey - my0
                w0v = jnp.exp(dx * dx * cx0 + dy * dy * cy0) * mf
                dx = ex - mx1
                dy = ey - my1
                w1v = jnp.exp(dx * dx * cx1 + dy * dy * cy1) * mf
                dx = ex - mx2
                dy = ey - my2
                w2v = jnp.exp(dx * dx * cx2 + dy * dy * cy2) * mf
                for j in range(16):
                    i = q * 16 + j
                    mv = (rows[i, pl.ds(0, 16)] * w0v[j]
                          + rows[i, pl.ds(16, 16)] * w1v[j]
                          + rows[i, pl.ds(32, 16)] * w2v[j])
                    msg[i, :] = mv
                return 0
            lax.fori_loop(0, G // 16, ed16, 0)

        def _drain_scatters():
            pltpu.make_async_copy(msg0, acc.at[sidx0], ss0).wait()
            pltpu.make_async_copy(msg1, acc.at[sidx1], ss1).wait()
            if with_cnt:
                pltpu.make_async_copy(mfb0, acc_cnt.at[sidx0], cs0).wait()
                pltpu.make_async_copy(mfb1, acc_cnt.at[sidx1], cs1).wait()

        NB = C // G

        def chunk_body(ch, _):
            ci = s * NCHUNK + ch
            pltpu.sync_copy(edata_hbm.at[ci], stage_all)

            pltpu.async_copy(y_hbm.at[stage_all.at[1, pl.ds(0, G)]], rows0, gs0)

            def pair(p, _):
                b0 = 2 * p * G
                b1 = b0 + G
                pltpu.async_copy(y_hbm.at[stage_all.at[1, pl.ds(b1, G)]], rows1, gs1)
                pltpu.make_async_copy(y_hbm.at[stage_all.at[1, pl.ds(0, G)]],
                                      rows0, gs0).wait()

                @pl.when(p > 0)
                def _():
                    pltpu.make_async_copy(msg0, acc.at[sidx0], ss0).wait()
                    if with_cnt:
                        pltpu.make_async_copy(mfb0, acc_cnt.at[sidx0], cs0).wait()

                _compute_block(b0, rows0, msg0, sidx0, mfb0 if with_cnt else None)
                pltpu.async_copy(msg0, acc.at[sidx0], ss0, add=True)
                if with_cnt:
                    pltpu.async_copy(mfb0, acc_cnt.at[sidx0], cs0, add=True)

                @pl.when(p < NB // 2 - 1)
                def _():
                    pltpu.async_copy(y_hbm.at[stage_all.at[1, pl.ds(b1 + G, G)]],
                                     rows0, gs0)

                pltpu.make_async_copy(y_hbm.at[stage_all.at[1, pl.ds(0, G)]],
                                      rows1, gs1).wait()

                @pl.when(p > 0)
                def _():
                    pltpu.make_async_copy(msg1, acc.at[sidx1], ss1).wait()
                    if with_cnt:
                        pltpu.make_async_copy(mfb1, acc_cnt.at[sidx1], cs1).wait()

                _compute_block(b1, rows1, msg1, sidx1, mfb1 if with_cnt else None)
                pltpu.async_copy(msg1, acc.at[sidx1], ss1, add=True)
                if with_cnt:
                    pltpu.async_copy(mfb1, acc_cnt.at[sidx1], cs1, add=True)
                return 0

            lax.fori_loop(0, NB // 2, pair, 0)
            _drain_scatters()
            return 0

        lax.fori_loop(0, NCHUNK, chunk_body, 0)

        plsc.subcore_barrier()

        def wblock(jb, _):
            r0 = s * RPT + jb * 512
            pltpu.sync_copy(acc.at[pl.ds(r0, 512)], wbuf)
            if with_cnt:
                pltpu.sync_copy(acc_cnt.at[pl.ds(r0, 512)], cbuf)
                pltpu.sync_copy(cbuf, cnt_out.at[c, pl.ds(r0, 512)])
            else:
                pltpu.sync_copy(cnt_hbm.at[c, pl.ds(r0, 512)], cbuf)

            def wg(q, _):
                cv = jnp.maximum(cbuf[pl.ds(q * 16, 16)], 1.0)
                for j in range(16):
                    i = q * 16 + j
                    wbuf[i, :] = wbuf[i, :] / cv[j]
                return 0
            lax.fori_loop(0, 32, wg, 0)
            pltpu.sync_copy(wbuf, aggm_out.at[c, pl.ds(r0, 512)])
            return 0
        lax.fori_loop(0, 10, wblock, 0)

        r0 = s * RPT + 5120
        pltpu.sync_copy(acc.at[pl.ds(r0, 8)], wbuf.at[pl.ds(0, 8)])
        if with_cnt:
            pltpu.sync_copy(acc_cnt.at[pl.ds(r0, 8)], cbuf.at[pl.ds(0, 8)])
            pltpu.sync_copy(cbuf.at[pl.ds(0, 8)], cnt_out.at[c, pl.ds(r0, 8)])
        else:
            pltpu.sync_copy(cnt_hbm.at[c, pl.ds(r0, 8)], cbuf.at[pl.ds(0, 8)])
        cv = jnp.maximum(cbuf[pl.ds(0, 16)], 1.0)
        for j in range(8):
            wbuf[j, :] = wbuf[j, :] / cv[j]
        pltpu.sync_copy(wbuf.at[pl.ds(0, 8)], aggm_out.at[c, pl.ds(r0, 8)])

    return body


def _sc_scratch(with_cnt):
    scr = [
        pltpu.VMEM((4, C), jnp.int32),
        pltpu.VMEM((G,), jnp.int32),
        pltpu.VMEM((G,), jnp.int32),
    ]
    if with_cnt:
        scr += [
            pltpu.VMEM((G,), jnp.float32),
            pltpu.VMEM((G,), jnp.float32),
        ]
    scr += [
        pltpu.VMEM((G, 48), jnp.float32),
        pltpu.VMEM((G, 48), jnp.float32),
        pltpu.VMEM((G, 16), jnp.float32),
        pltpu.VMEM((G, 16), jnp.float32),
        pltpu.VMEM((16,), jnp.float32),
        pltpu.VMEM((512, 16), jnp.float32),
        pltpu.VMEM((512,), jnp.float32),
        pltpu.VMEM_SHARED((H2, 16), jnp.float32),
    ]
    if with_cnt:
        scr += [pltpu.VMEM_SHARED((H2,), jnp.float32)]
    scr += [pltpu.SemaphoreType.DMA] * (6 if with_cnt else 4)
    return scr


_sc_layer1 = pl.kernel(
    _make_sc_body(True),
    out_type=[jax.ShapeDtypeStruct((2, H2, 16), jnp.float32),
              jax.ShapeDtypeStruct((2, H2), jnp.float32)],
    mesh=plsc.VectorSubcoreMesh(core_axis_name="c", subcore_axis_name="s"),
    compiler_params=pltpu.CompilerParams(use_tc_tiling_on_sc=False, needs_layout_passes=False),
    scratch_types=_sc_scratch(True),
)

_sc_layer2 = pl.kernel(
    _make_sc_body(False),
    out_type=jax.ShapeDtypeStruct((2, H2, 16), jnp.float32),
    mesh=plsc.VectorSubcoreMesh(core_axis_name="c", subcore_axis_name="s"),
    compiler_params=pltpu.CompilerParams(use_tc_tiling_on_sc=False, needs_layout_passes=False),
    scratch_types=_sc_scratch(False),
)


def kernel(data, edges, edge_vectors, g0, mu0, sigma0, root0, bias0,
           g1, mu1, sigma1, root1, bias1, fc_w, fc_b):
    f32 = jnp.float32
    src = edges[0]
    dst = edges[1]
    evx = jax.lax.bitcast_convert_type(edge_vectors[:, 0], jnp.int32)
    evy = jax.lax.bitcast_convert_type(edge_vectors[:, 1], jnp.int32)
    edata = jnp.stack([dst.reshape(NCHG, C), src.reshape(NCHG, C),
                       evx.reshape(NCHG, C), evy.reshape(NCHG, C)], axis=1)
    data_p = jnp.pad(data, ((0, NP - N), (0, 0)))
    x_fold = data_p.reshape(NF, 128)

    eye8 = jnp.eye(8, dtype=f32)
    gb0 = jnp.kron(eye8, g0)
    rb0 = jnp.kron(eye8, root0)
    gb1 = jnp.kron(eye8, g1)
    rb1 = jnp.kron(eye8, root1)
    bb0 = jnp.tile(bias0, 8)
    bb1 = jnp.tile(bias1, 8)
    wb2 = jnp.concatenate([jnp.kron(eye8, fc_w), jnp.kron(eye8, fc_w[:, ::-1])], axis=1)
    fb2 = jnp.concatenate([jnp.tile(fc_b, 8), jnp.tile(fc_b[::-1], 8)])

    def params_of(mu, sigma):
        csig = -0.5 / (1e-15 + sigma.astype(f32) ** 2)
        return jnp.concatenate([mu.astype(f32).reshape(-1), csig.reshape(-1),
                                jnp.zeros((4,), f32)])

    prm0 = params_of(mu0, sigma0)
    prm1 = params_of(mu1, sigma1)

    y1, r1 = _dense(x_fold, gb0, rb0)
    aggm1, cnt = _sc_layer1(y1.reshape(NP, 48), edata, prm0)
    aggm1_fold = aggm1.reshape(NF, 128)

    y2, r2 = _mid(aggm1_fold, r1, bb0, gb1, rb1)
    aggm2 = _sc_layer2(y2.reshape(NP, 48), edata, prm1, cnt)
    aggm2_fold = aggm2.reshape(NF, 128)

    ls, lg = _final(aggm2_fold, r2, bb1, wb2, fb2)
    log_sm = ls.reshape(NP, 2)[:N]
    nll = lg.reshape(NP, 2)[:N, 0:1]
    return (log_sm, nll)


# speedup vs baseline: 1.2366x; 1.0092x over previous
import jax
import jax.numpy as jnp
from jax import lax
from jax.experimental import pallas as pl
from jax.experimental.pallas import tpu as pltpu
from jax.experimental.pallas import tpu_sc as plsc

N = 163842
E = 983040
H2 = 82048
NP = 2 * H2
RPT = H2 // 16
NF = NP // 8
ET = E // 16
C = 4096
NCHUNK = ET // C
NCHG = E // C
G = 128


def _dense_body(x_ref, g_ref, root_ref, y_ref, r_ref):
    x = x_ref[...]
    y_ref[...] = jnp.dot(x, g_ref[...], preferred_element_type=jnp.float32)
    r_ref[...] = jnp.dot(x, root_ref[...], preferred_element_type=jnp.float32)


def _dense(x_fold, g_big, root_big):
    blk = 2048
    return pl.pallas_call(
        _dense_body,
        grid=(pl.cdiv(NF, blk),),
        in_specs=[
            pl.BlockSpec((blk, 128), lambda i: (i, 0)),
            pl.BlockSpec((128, 384), lambda i: (0, 0)),
            pl.BlockSpec((128, 128), lambda i: (0, 0)),
        ],
        out_specs=[
            pl.BlockSpec((blk, 384), lambda i: (i, 0)),
            pl.BlockSpec((blk, 128), lambda i: (i, 0)),
        ],
        out_shape=[
            jax.ShapeDtypeStruct((NF, 384), jnp.float32),
            jax.ShapeDtypeStruct((NF, 128), jnp.float32),
        ],
    )(x_fold, g_big, root_big)


def _mid_body(aggm_ref, r_ref, bias_ref, g_ref, root_ref, y_ref, r2_ref):
    x = jnp.maximum(aggm_ref[...] + r_ref[...] + bias_ref[...], 0.0)
    y_ref[...] = jnp.dot(x, g_ref[...], preferred_element_type=jnp.float32)
    r2_ref[...] = jnp.dot(x, root_ref[...], preferred_element_type=jnp.float32)


def _mid(aggm_fold, r_fold, bias_big, g_big, root_big):
    blk = 2048
    return pl.pallas_call(
        _mid_body,
        grid=(pl.cdiv(NF, blk),),
        in_specs=[
            pl.BlockSpec((blk, 128), lambda i: (i, 0)),
            pl.BlockSpec((blk, 128), lambda i: (i, 0)),
            pl.BlockSpec((128,), lambda i: (0,)),
            pl.BlockSpec((128, 384), lambda i: (0, 0)),
            pl.BlockSpec((128, 128), lambda i: (0, 0)),
        ],
        out_specs=[
            pl.BlockSpec((blk, 384), lambda i: (i, 0)),
            pl.BlockSpec((blk, 128), lambda i: (i, 0)),
        ],
        out_shape=[
            jax.ShapeDtypeStruct((NF, 384), jnp.float32),
            jax.ShapeDtypeStruct((NF, 128), jnp.float32),
        ],
    )(aggm_fold, r_fold, bias_big, g_big, root_big)


def _final_body(aggm_ref, r_ref, bias_ref, w_ref, b_ref, ls_ref, lg_ref):
    x = jnp.maximum(aggm_ref[...] + r_ref[...] + bias_ref[...], 0.0)
    ll = jnp.dot(x, w_ref[...], preferred_element_type=jnp.float32) + b_ref[...]
    lo = ll[:, 0:16]
    lsw = ll[:, 16:32]
    mx = jnp.maximum(lo, lsw)
    lse = mx + jnp.log(jnp.exp(lo - mx) + jnp.exp(lsw - mx))
    ls_ref[...] = lo - lse
    lg_ref[...] = lo


def _final(aggm_fold, r_fold, bias_big, w_big2, b_big2):
    blk = 2048
    return pl.pallas_call(
        _final_body,
        grid=(pl.cdiv(NF, blk),),
        in_specs=[
            pl.BlockSpec((blk, 128), lambda i: (i, 0)),
            pl.BlockSpec((blk, 128), lambda i: (i, 0)),
            pl.BlockSpec((128,), lambda i: (0,)),
            pl.BlockSpec((128, 32), lambda i: (0, 0)),
            pl.BlockSpec((32,), lambda i: (0,)),
        ],
        out_specs=[
            pl.BlockSpec((blk, 16), lambda i: (i, 0)),
            pl.BlockSpec((blk, 16), lambda i: (i, 0)),
        ],
        out_shape=[
            jax.ShapeDtypeStruct((NF, 16), jnp.float32),
            jax.ShapeDtypeStruct((NF, 16), jnp.float32),
        ],
    )(aggm_fold, r_fold, bias_big, w_big2, b_big2)


def _make_sc_body(with_cnt):
    def body(*args):
        if with_cnt:
            (y_hbm, edata_hbm, prm_hbm, aggm_out, cnt_out,
             stage_all, sidx0, sidx1, mfb0, mfb1, rows0, rows1, msg0, msg1,
             pbuf, wbuf, cbuf, acc, acc_cnt,
             gs0, gs1, ss0, ss1, cs0, cs1) = args
        else:
            (y_hbm, edata_hbm, prm_hbm, cnt_hbm, aggm_out,
             stage_all, sidx0, sidx1, rows0, rows1, msg0, msg1,
             pbuf, wbuf, cbuf, acc,
             gs0, gs1, ss0, ss1) = args
        c = lax.axis_index("c")
        s = lax.axis_index("s")
        base = c * H2

        pltpu.sync_copy(prm_hbm, pbuf)
        pv = pbuf[pl.ds(0, 16)]
        mx0, my0, mx1, my1, mx2, my2 = pv[0], pv[1], pv[2], pv[3], pv[4], pv[5]
        cx0, cy0, cx1, cy1, cx2, cy2 = pv[6], pv[7], pv[8], pv[9], pv[10], pv[11]

        zf = jnp.zeros((16,), jnp.float32)

        def fill16(i, _):
            wbuf[i, :] = zf
            return 0
        lax.fori_loop(0, 512, fill16, 0)

        def fill1(i, _):
            cbuf[pl.ds(i * 16, 16)] = zf
            return 0
        lax.fori_loop(0, 32, fill1, 0)

        def zacc(j, _):
            pltpu.sync_copy(wbuf, acc.at[pl.ds(s * RPT + j * 512, 512)])
            if with_cnt:
                pltpu.sync_copy(cbuf, acc_cnt.at[pl.ds(s * RPT + j * 512, 512)])
            return 0
        lax.fori_loop(0, 10, zacc, 0)
        pltpu.sync_copy(wbuf.at[pl.ds(0, 8)], acc.at[pl.ds(s * RPT + 5120, 8)])
        if with_cnt:
            pltpu.sync_copy(cbuf.at[pl.ds(0, 8)],
                            acc_cnt.at[pl.ds(s * RPT + 5120, 8)])

        plsc.subcore_barrier()

        def _compute_block(boff, rows, msg, sidx, mfb):
            def ed16(q, _):
                off = boff + q * 16
                d = stage_all[0, pl.ds(off, 16)]
                dl = d - base
                inr = (dl >= 0) & (dl < H2)
                mf = jnp.where(inr, 1.0, 0.0).astype(jnp.float32)
                sidx[pl.ds(q * 16, 16)] = jnp.where(inr, dl, 0)
                if with_cnt:
                    mfb[pl.ds(q * 16, 16)] = mf
                ex = plsc.bitcast(stage_all[2, pl.ds(off, 16)], jnp.float32)
                ey = plsc.bitcast(stage_all[3, pl.ds(off, 16)], jnp.float32)
                dx = ex - mx0
                dy = ey - my0
                w0v = jnp.exp(dx * dx * cx0 + dy * dy * cy0) * mf
                dx = ex - mx1
                dy = ey - my1
                w1v = jnp.exp(dx * dx * cx1 + dy * dy * cy1) * mf
                dx = ex - mx2
                dy = ey - my2
                w2v = jnp.exp(dx * dx * cx2 + dy * dy * cy2) * mf
                for j in range(16):
                    i = q * 16 + j
                    mv = (rows[i, pl.ds(0, 16)] * w0v[j]
                          + rows[i, pl.ds(16, 16)] * w1v[j]
                          + rows[i, pl.ds(32, 16)] * w2v[j])
                    msg[i, :] = mv
                return 0
            lax.fori_loop(0, G // 16, ed16, 0)

        def _drain_scatters():
            pltpu.make_async_copy(msg0, acc.at[sidx0], ss0).wait()
            pltpu.make_async_copy(msg1, acc.at[sidx1], ss1).wait()
            if with_cnt:
                pltpu.make_async_copy(mfb0, acc_cnt.at[sidx0], cs0).wait()
                pltpu.make_async_copy(mfb1, acc_cnt.at[sidx1], cs1).wait()

        NB = C // G

        def chunk_body(ch, _):
            ci = s * NCHUNK + ch
            pltpu.sync_copy(edata_hbm.at[ci], stage_all)

            pltpu.async_copy(y_hbm.at[stage_all.at[1, pl.ds(0, G)]], rows0, gs0)

            def pair(p, _):
                b0 = 2 * p * G
                b1 = b0 + G
                pltpu.async_copy(y_hbm.at[stage_all.at[1, pl.ds(b1, G)]], rows1, gs1)
                pltpu.make_async_copy(y_hbm.at[stage_all.at[1, pl.ds(0, G)]],
                                      rows0, gs0).wait()

                @pl.when(p > 0)
                def _():
                    pltpu.make_async_copy(msg0, acc.at[sidx0], ss0).wait()
                    if with_cnt:
                        pltpu.make_async_copy(mfb0, acc_cnt.at[sidx0], cs0).wait()

                _compute_block(b0, rows0, msg0, sidx0, mfb0 if with_cnt else None)
                pltpu.async_copy(msg0, acc.at[sidx0], ss0, add=True)
                if with_cnt:
                    pltpu.async_copy(mfb0, acc_cnt.at[sidx0], cs0, add=True)

                @pl.when(p < NB // 2 - 1)
                def _():
                    pltpu.async_copy(y_hbm.at[stage_all.at[1, pl.ds(b1 + G, G)]],
                                     rows0, gs0)

                pltpu.make_async_copy(y_hbm.at[stage_all.at[1, pl.ds(0, G)]],
                                      rows1, gs1).wait()

                @pl.when(p > 0)
                def _():
                    pltpu.make_async_copy(msg1, acc.at[sidx1], ss1).wait()
                    if with_cnt:
                        pltpu.make_async_copy(mfb1, acc_cnt.at[sidx1], cs1).wait()

                _compute_block(b1, rows1, msg1, sidx1, mfb1 if with_cnt else None)
                pltpu.async_copy(msg1, acc.at[sidx1], ss1, add=True)
                if with_cnt:
                    pltpu.async_copy(mfb1, acc_cnt.at[sidx1], cs1, add=True)
                return 0

            lax.fori_loop(0, NB // 2, pair, 0)
            _drain_scatters()
            return 0

        lax.fori_loop(0, NCHUNK, chunk_body, 0)

        plsc.subcore_barrier()

        def wblock(jb, _):
            r0 = s * RPT + jb * 512
            pltpu.sync_copy(acc.at[pl.ds(r0, 512)], wbuf)
            if with_cnt:
                pltpu.sync_copy(acc_cnt.at[pl.ds(r0, 512)], cbuf)
                pltpu.sync_copy(cbuf, cnt_out.at[c, pl.ds(r0, 512)])
            else:
                pltpu.sync_copy(cnt_hbm.at[c, pl.ds(r0, 512)], cbuf)

            def wg(q, _):
                cv = jnp.maximum(cbuf[pl.ds(q * 16, 16)], 1.0)
                for j in range(16):
                    i = q * 16 + j
                    wbuf[i, :] = wbuf[i, :] / cv[j]
                return 0
            lax.fori_loop(0, 32, wg, 0)
            pltpu.sync_copy(wbuf, aggm_out.at[c, pl.ds(r0, 512)])
            return 0
        lax.fori_loop(0, 10, wblock, 0)

        r0 = s * RPT + 5120
        pltpu.sync_copy(acc.at[pl.ds(r0, 8)], wbuf.at[pl.ds(0, 8)])
        if with_cnt:
            pltpu.sync_copy(acc_cnt.at[pl.ds(r0, 8)], cbuf.at[pl.ds(0, 8)])
            pltpu.sync_copy(cbuf.at[pl.ds(0, 8)], cnt_out.at[c, pl.ds(r0, 8)])
        else:
            pltpu.sync_copy(cnt_hbm.at[c, pl.ds(r0, 8)], cbuf.at[pl.ds(0, 8)])
        cv = jnp.maximum(cbuf[pl.ds(0, 16)], 1.0)
        for j in range(8):
            wbuf[j, :] = wbuf[j, :] / cv[j]
        pltpu.sync_copy(wbuf.at[pl.ds(0, 8)], aggm_out.at[c, pl.ds(r0, 8)])

    return body


def _sc_scratch(with_cnt):
    scr = [
        pltpu.VMEM((4, C), jnp.int32),
        pltpu.VMEM((G,), jnp.int32),
        pltpu.VMEM((G,), jnp.int32),
    ]
    if with_cnt:
        scr += [
            pltpu.VMEM((G,), jnp.float32),
            pltpu.VMEM((G,), jnp.float32),
        ]
    scr += [
        pltpu.VMEM((G, 48), jnp.float32),
        pltpu.VMEM((G, 48), jnp.float32),
        pltpu.VMEM((G, 16), jnp.float32),
        pltpu.VMEM((G, 16), jnp.float32),
        pltpu.VMEM((16,), jnp.float32),
        pltpu.VMEM((512, 16), jnp.float32),
        pltpu.VMEM((512,), jnp.float32),
        pltpu.VMEM_SHARED((H2, 16), jnp.float32),
    ]
    if with_cnt:
        scr += [pltpu.VMEM_SHARED((H2,), jnp.float32)]
    scr += [pltpu.SemaphoreType.DMA] * (6 if with_cnt else 4)
    return scr


_sc_layer1 = pl.kernel(
    _make_sc_body(True),
    out_type=[jax.ShapeDtypeStruct((2, H2, 16), jnp.float32),
              jax.ShapeDtypeStruct((2, H2), jnp.float32)],
    mesh=plsc.VectorSubcoreMesh(core_axis_name="c", subcore_axis_name="s"),
    compiler_params=pltpu.CompilerParams(use_tc_tiling_on_sc=False, needs_layout_passes=False),
    scratch_types=_sc_scratch(True),
)

_sc_layer2 = pl.kernel(
    _make_sc_body(False),
    out_type=jax.ShapeDtypeStruct((2, H2, 16), jnp.float32),
    mesh=plsc.VectorSubcoreMesh(core_axis_name="c", subcore_axis_name="s"),
    compiler_params=pltpu.CompilerParams(use_tc_tiling_on_sc=False, needs_layout_passes=False),
    scratch_types=_sc_scratch(False),
)


def kernel(data, edges, edge_vectors, g0, mu0, sigma0, root0, bias0,
           g1, mu1, sigma1, root1, bias1, fc_w, fc_b):
    f32 = jnp.float32
    src = edges[0]
    dst = edges[1]
    evx = jax.lax.bitcast_convert_type(edge_vectors[:, 0], jnp.int32)
    evy = jax.lax.bitcast_convert_type(edge_vectors[:, 1], jnp.int32)
    edata = jnp.stack([dst.reshape(NCHG, C), src.reshape(NCHG, C),
                       evx.reshape(NCHG, C), evy.reshape(NCHG, C)], axis=1)
    data_p = jnp.pad(data, ((0, NP - N), (0, 0)))
    x_fold = data_p.reshape(NF, 128)

    eye8 = jnp.eye(8, dtype=f32)
    gb0 = jnp.kron(eye8, g0)
    rb0 = jnp.kron(eye8, root0)
    gb1 = jnp.kron(eye8, g1)
    rb1 = jnp.kron(eye8, root1)
    bb0 = jnp.tile(bias0, 8)
    bb1 = jnp.tile(bias1, 8)
    wb2 = jnp.concatenate([jnp.kron(eye8, fc_w), jnp.kron(eye8, fc_w[:, ::-1])], axis=1)
    fb2 = jnp.concatenate([jnp.tile(fc_b, 8), jnp.tile(fc_b[::-1], 8)])

    def params_of(mu, sigma):
        csig = -0.5 / (1e-15 + sigma.astype(f32) ** 2)
        return jnp.concatenate([mu.astype(f32).reshape(-1), csig.reshape(-1),
                                jnp.zeros((4,), f32)])

    prm0 = params_of(mu0, sigma0)
    prm1 = params_of(mu1, sigma1)

    y1, r1 = _dense(x_fold, gb0, rb0)
    aggm1, cnt = _sc_layer1(y1.reshape(NP, 48), edata, prm0)
    aggm1_fold = aggm1.reshape(NF, 128)

    y2, r2 = _mid(aggm1_fold, r1, bb0, gb1, rb1)
    aggm2 = _sc_layer2(y2.reshape(NP, 48), edata, prm1, cnt)
    aggm2_fold = aggm2.reshape(NF, 128)

    ls, lg = _final(aggm2_fold, r2, bb1, wb2, fb2)
    log_sm = ls.reshape(NP, 2)[:N]
    nll = lg.reshape(NP, 2)[:N, 0:1]
    return (log_sm, nll)
